# baseline (speedup 1.0000x reference)
import functools

import jax
import jax.numpy as jnp
from jax import lax
from jax.experimental import pallas as pl
from jax.experimental.pallas import tpu as pltpu
from jax.experimental.pallas import tpu_sc as plsc

_N = 10000
_NP = 10240
_E = 320000
_F = 128
_W = 128
_NC = 2
_NS = 16
_NW = _NC * _NS
_CH = 64
_NB = 4
_NI = 8
_CHUNKS = 160
_EPW = _CHUNKS * _CH
_NE = _NW * _EPW
_RPS = _NP // _NS
_BLK = 1024
_B = 1024
_BPW = _B // _NW
_ECH = 1280

_mesh = plsc.VectorSubcoreMesh(core_axis_name="c", subcore_axis_name="s")


@functools.partial(
    pl.kernel,
    out_type=jax.ShapeDtypeStruct((_NW, _NP), jnp.float32),
    mesh=_mesh,
    scratch_types=[
        pltpu.VMEM((_NP,), jnp.float32),
        pltpu.VMEM((_ECH,), jnp.int32),
    ],
    compiler_params=pltpu.CompilerParams(needs_layout_passes=False),
)
def _deg_kernel(dst_hbm, out_hbm, hist, dst_v):
    c = lax.axis_index("c")
    s = lax.axis_index("s")
    wid = c * _NS + s
    zv = jnp.zeros((16,), jnp.float32)

    def zloop(i, carry):
        hist[pl.ds(i * 16, 16)] = zv
        return carry

    lax.fori_loop(0, _NP // 16, zloop, 0)

    ones = jnp.ones((16,), jnp.float32)
    base = wid * _EPW

    def outer(t, carry):
        pltpu.sync_copy(dst_hbm.at[pl.ds(base + t * _ECH, _ECH)], dst_v)

        def inner(j, carry2):
            idx = dst_v[pl.ds(j * 16, 16)]
            plsc.addupdate_scatter(hist, [idx], ones)
            return carry2

        lax.fori_loop(0, _ECH // 16, inner, 0)
        return carry

    lax.fori_loop(0, _EPW // _ECH, outer, 0)
    pltpu.sync_copy(hist, out_hbm.at[wid])


@functools.partial(
    pl.kernel,
    out_type=jax.ShapeDtypeStruct((_NC, _NP, _W), jnp.float32),
    mesh=_mesh,
    scratch_types=[
        pltpu.VMEM_SHARED((_NP, _W), jnp.float32),
        [pltpu.VMEM((_CH,), jnp.int32) for _ in range(_NI)],
        [pltpu.VMEM((_CH,), jnp.int32) for _ in range(_NI)],
        [pltpu.VMEM((_CH, _W), jnp.float32) for _ in range(_NB)],
        [pltpu.SemaphoreType.DMA for _ in range(_NI)],
        [pltpu.SemaphoreType.DMA for _ in range(_NI)],
        [pltpu.SemaphoreType.DMA for _ in range(_NB)],
    ],
)
def _scatter(g_hbm, src_hbm, dst_hbm, out_hbm, acc, srcb, dstb, rows,
             isem, dsem, gsem):
    c = lax.axis_index("c")
    s = lax.axis_index("s")
    wid = c * _NS + s

    zv = jnp.zeros((16,), jnp.float32)
    for i in range(16):
        for j in range(_W // 16):
            rows[0][i, 16 * j:16 * (j + 1)] = zv

    def zloop(k, carry):
        pltpu.sync_copy(rows[0].at[pl.ds(0, 16)],
                        acc.at[pl.ds(s * _RPS + k * 16, 16)])
        return carry

    lax.fori_loop(0, _RPS // 16, zloop, 0)

    for a in range(_NI):
        pltpu.async_copy(src_hbm.at[wid, a], srcb[a], isem[a])
        pltpu.async_copy(dst_hbm.at[wid, a], dstb[a], dsem[a])
    for b in range(_NB):
        pltpu.make_async_copy(src_hbm.at[wid, b], srcb[b], isem[b]).wait()
        pltpu.async_copy(g_hbm.at[srcb[b]], rows[b], gsem[b])
    plsc.subcore_barrier()

    def body(i, carry):
        for k in range(_NI):
            t = _NI * i + k
            b = k % _NB
            pltpu.make_async_copy(g_hbm.at[srcb[k]], rows[b], gsem[b]).wait()
            pltpu.make_async_copy(dst_hbm.at[wid, t], dstb[k], dsem[k]).wait()
            pltpu.sync_copy(rows[b], acc.at[dstb[k]], add=True)
            ti = t + _NI

            @pl.when(ti < _CHUNKS)
            def _():
                pltpu.async_copy(src_hbm.at[wid, ti], srcb[k], isem[k])
                pltpu.async_copy(dst_hbm.at[wid, ti], dstb[k], dsem[k])

            tg = t + _NB
            kg = (k + _NB) % _NI

            @pl.when(tg < _CHUNKS)
            def _():
                pltpu.make_async_copy(
                    src_hbm.at[wid, tg], srcb[kg], isem[kg]).wait()
                pltpu.async_copy(g_hbm.at[srcb[kg]], rows[b], gsem[b])

        return carry

    lax.fori_loop(0, _CHUNKS // _NI, body, 0)
    plsc.subcore_barrier()
    pltpu.sync_copy(acc.at[pl.ds(s * _RPS, _RPS)],
                    out_hbm.at[c, pl.ds(s * _RPS, _RPS)])


@functools.partial(
    pl.kernel,
    out_type=jax.ShapeDtypeStruct((_B, _W), jnp.float32),
    mesh=_mesh,
    scratch_types=[
        pltpu.VMEM((_BPW,), jnp.int32),
        pltpu.VMEM((_BPW, _W), jnp.float32),
        pltpu.SemaphoreType.DMA,
    ],
)
def _gather_rows(h_hbm, idx_hbm, out_hbm, idx_v, rows_v, sem):
    c = lax.axis_index("c")
    s = lax.axis_index("s")
    wid = c * _NS + s
    pltpu.sync_copy(idx_hbm.at[pl.ds(wid * _BPW, _BPW)], idx_v)
    pltpu.async_copy(h_hbm.at[idx_v], rows_v, sem).wait()
    pltpu.sync_copy(rows_v, out_hbm.at[pl.ds(wid * _BPW, _BPW)])


def _scale_in_body(degp_ref, x_ref, w_ref, g_ref, dinv_ref):
    deg = jnp.sum(degp_ref[...], axis=0)[:, None] + 1.0
    dinv = lax.rsqrt(deg)
    i = pl.program_id(0)
    rows = i * _BLK + lax.broadcasted_iota(jnp.int32, (_BLK, 1), 0)
    h = jnp.dot(x_ref[...], w_ref[...], preferred_element_type=jnp.float32)
    g_ref[...] = jnp.where(rows < _N, h * dinv, 0.0)
    dinv_ref[...] = dinv


_scale_in = pl.pallas_call(
    _scale_in_body,
    grid=(_NP // _BLK,),
    in_specs=[
        pl.BlockSpec((_NW, _BLK), lambda i: (0, i)),
        pl.BlockSpec((_BLK, _F), lambda i: (i, 0)),
        pl.BlockSpec((_F, _W), lambda i: (0, 0)),
    ],
    out_specs=[
        pl.BlockSpec((_BLK, _W), lambda i: (i, 0)),
        pl.BlockSpec((_BLK, 1), lambda i: (i, 0)),
    ],
    out_shape=[
        jax.ShapeDtypeStruct((_NP, _W), jnp.float32),
        jax.ShapeDtypeStruct((_NP, 1), jnp.float32),
    ],
)


def _combine_matmul_body(p_ref, g_ref, dinv_ref, b_ref, w_ref, out_ref):
    i = pl.program_id(0)
    p = p_ref[...]
    dinv = dinv_ref[...]
    u = dinv * (p[0] + p[1] + g_ref[...]) + b_ref[...]
    rows = i * _BLK + lax.broadcasted_iota(jnp.int32, (_BLK, 1), 0)
    v = jnp.where(rows < _N, jnp.maximum(u, 0.0), 0.0)
    out_ref[...] = jnp.dot(
        v, w_ref[...], preferred_element_type=jnp.float32) * dinv


_combine_matmul = pl.pallas_call(
    _combine_matmul_body,
    grid=(_NP // _BLK,),
    in_specs=[
        pl.BlockSpec((2, _BLK, _W), lambda i: (0, i, 0)),
        pl.BlockSpec((_BLK, _W), lambda i: (i, 0)),
        pl.BlockSpec((_BLK, 1), lambda i: (i, 0)),
        pl.BlockSpec((1, _W), lambda i: (0, 0)),
        pl.BlockSpec((_W, _W), lambda i: (0, 0)),
    ],
    out_specs=pl.BlockSpec((_BLK, _W), lambda i: (i, 0)),
    out_shape=jax.ShapeDtypeStruct((_NP, _W), jnp.float32),
)


def _combine_out_body(p_ref, g_ref, dinv_ref, b_ref, out_ref):
    p = p_ref[...]
    out_ref[...] = dinv_ref[...] * (p[0] + p[1] + g_ref[...]) + b_ref[...]


_combine_out = pl.pallas_call(
    _combine_out_body,
    grid=(_NP // _BLK,),
    in_specs=[
        pl.BlockSpec((2, _BLK, _W), lambda i: (0, i, 0)),
        pl.BlockSpec((_BLK, _W), lambda i: (i, 0)),
        pl.BlockSpec((_BLK, 1), lambda i: (i, 0)),
        pl.BlockSpec((1, _W), lambda i: (0, 0)),
    ],
    out_specs=pl.BlockSpec((_BLK, _W), lambda i: (i, 0)),
    out_shape=jax.ShapeDtypeStruct((_NP, _W), jnp.float32),
)


def _head_body(hg_ref, wl_ref, bl_ref, lsm_ref, sm_ref):
    z = jnp.dot(hg_ref[...], wl_ref[...],
                preferred_element_type=jnp.float32) + bl_ref[...]
    m = jnp.max(z, axis=1, keepdims=True)
    zc = z - m
    e = jnp.exp(zc)
    ssum = jnp.sum(e, axis=1, keepdims=True)
    sm_ref[...] = e / ssum
    lsm_ref[...] = zc - jnp.log(ssum)


_head = pl.pallas_call(
    _head_body,
    out_shape=[
        jax.ShapeDtypeStruct((_B, 2), jnp.float32),
        jax.ShapeDtypeStruct((_B, 2), jnp.float32),
    ],
)


def kernel(x, edge_index, relevant_batch_indices, labels,
           W1, b1, W2, b2, W3, b3, Wlin, blin):
    src = jnp.asarray(edge_index[0], jnp.int32)
    dst = jnp.asarray(edge_index[1], jnp.int32)
    padv = jnp.full((_NE - _E,), _N, jnp.int32)
    src_p = jnp.concatenate([src, padv])
    dst_p = jnp.concatenate([dst, padv])
    x_p = jnp.pad(x, ((0, _NP - _N), (0, 0)))

    W2p = jnp.pad(W2, ((0, 0), (0, _W - W2.shape[1])))
    b2p = jnp.pad(b2, (0, _W - b2.shape[0])).reshape(1, -1)
    W3p = jnp.pad(W3, ((0, _W - W3.shape[0]), (0, _W - W3.shape[1])))
    b3p = jnp.pad(b3, (0, _W - b3.shape[0])).reshape(1, -1)
    Wlp = jnp.pad(Wlin, ((0, _W - Wlin.shape[0]), (0, 0)))

    src3 = src_p.reshape(_NW, _CHUNKS, _CH)
    dst3 = dst_p.reshape(_NW, _CHUNKS, _CH)

    degp = _deg_kernel(dst_p)
    g1, dinv = _scale_in(degp, x_p, W1)
    p1 = _scatter(g1, src3, dst3)
    g2 = _combine_matmul(p1, g1, dinv, b1.reshape(1, -1), W2p)
    p2 = _scatter(g2, src3, dst3)
    g3 = _combine_matmul(p2, g2, dinv, b2p, W3p)
    p3 = _scatter(g3, src3, dst3)
    h3 = _combine_out(p3, g3, dinv, b3p)
    hg = _gather_rows(h3, jnp.asarray(relevant_batch_indices, jnp.int32))
    lsm, sm = _head(hg, Wlp, blin.reshape(1, -1))
    return (lsm, sm)


# speedup vs baseline: 1.7616x; 1.7616x over previous
import functools

import jax
import jax.numpy as jnp
from jax import lax
from jax.experimental import pallas as pl
from jax.experimental.pallas import tpu as pltpu
from jax.experimental.pallas import tpu_sc as plsc

_N = 10000
_NP = 10240
_E = 320000
_F = 128
_W = 128
_HQ = 32
_NC = 2
_NS = 16
_NW = _NC * _NS
_CH = 128
_CHUNKS = 80
_EPW = _CHUNKS * _CH
_NE = _NW * _EPW
_RPS = _NP // _NS
_BLK = 1024
_B = 1024
_BPW = _B // _NW
_ECH = 1280

_mesh = plsc.VectorSubcoreMesh(core_axis_name="c", subcore_axis_name="s")


@functools.partial(
    pl.kernel,
    out_type=jax.ShapeDtypeStruct((_NW, _NP), jnp.float32),
    mesh=_mesh,
    scratch_types=[
        pltpu.VMEM((_NP,), jnp.float32),
        pltpu.VMEM((_ECH,), jnp.int32),
    ],
    compiler_params=pltpu.CompilerParams(needs_layout_passes=False),
)
def _deg_kernel(dst_hbm, out_hbm, hist, dst_v):
    c = lax.axis_index("c")
    s = lax.axis_index("s")
    wid = c * _NS + s
    zv = jnp.zeros((16,), jnp.float32)

    def zloop(i, carry):
        hist[pl.ds(i * 16, 16)] = zv
        return carry

    lax.fori_loop(0, _NP // 16, zloop, 0)

    ones = jnp.ones((16,), jnp.float32)
    base = wid * _EPW

    def outer(t, carry):
        pltpu.sync_copy(dst_hbm.at[pl.ds(base + t * _ECH, _ECH)], dst_v)

        def inner(j, carry2):
            idx = dst_v[pl.ds(j * 16, 16)]
            plsc.addupdate_scatter(hist, [idx], ones)
            return carry2

        lax.fori_loop(0, _ECH // 16, inner, 0)
        return carry

    lax.fori_loop(0, _EPW // _ECH, outer, 0)
    pltpu.sync_copy(hist, out_hbm.at[wid])


def _make_scatter(width, tc_tiling):
    @functools.partial(
        pl.kernel,
        out_type=jax.ShapeDtypeStruct((_NC, _NP, width), jnp.float32),
        mesh=_mesh,
        scratch_types=[
            pltpu.VMEM_SHARED((_NP, width), jnp.float32),
            pltpu.VMEM((_CHUNKS, _CH), jnp.int32),
            pltpu.VMEM((_CH,), jnp.int32),
            pltpu.VMEM((_CH,), jnp.int32),
            pltpu.VMEM((_CH, width), jnp.float32),
            pltpu.VMEM((_CH, width), jnp.float32),
            pltpu.SemaphoreType.DMA,
            pltpu.SemaphoreType.DMA,
            pltpu.SemaphoreType.DMA,
            pltpu.SemaphoreType.DMA,
        ],
        compiler_params=pltpu.CompilerParams(use_tc_tiling_on_sc=tc_tiling),
    )
    def _scat(g_hbm, src_hbm, dst_hbm, out_hbm, acc, srcs, dstb0, dstb1,
              rows0, rows1, gsem0, gsem1, dsem0, dsem1):
        c = lax.axis_index("c")
        s = lax.axis_index("s")
        wid = c * _NS + s

        pltpu.sync_copy(src_hbm.at[wid], srcs)

        zv = jnp.zeros((16,), jnp.float32)
        for i in range(16):
            for j in range(width // 16):
                rows0[i, 16 * j:16 * (j + 1)] = zv

        def zloop(k, carry):
            pltpu.sync_copy(rows0.at[pl.ds(0, 16)],
                            acc.at[pl.ds(s * _RPS + k * 16, 16)])
            return carry

        lax.fori_loop(0, _RPS // 16, zloop, 0)
        plsc.subcore_barrier()

        pltpu.async_copy(dst_hbm.at[wid, 0], dstb0, dsem0)
        pltpu.async_copy(g_hbm.at[srcs.at[0]], rows0, gsem0)

        def body(i, carry):
            for b, rows, gsem, dstb, dsem, orows, ogsem, odstb, odsem in (
                (0, rows0, gsem0, dstb0, dsem0, rows1, gsem1, dstb1, dsem1),
                (1, rows1, gsem1, dstb1, dsem1, rows0, gsem0, dstb0, dsem0),
            ):
                t = 2 * i + b
                pltpu.make_async_copy(dst_hbm.at[wid, t], dstb, dsem).wait()
                pltpu.make_async_copy(g_hbm.at[srcs.at[t]], rows, gsem).wait()
                nt = t + 1

                @pl.when(nt < _CHUNKS)
                def _():
                    pltpu.async_copy(dst_hbm.at[wid, nt], odstb, odsem)
                    pltpu.async_copy(g_hbm.at[srcs.at[nt]], orows, ogsem)

                pltpu.sync_copy(rows, acc.at[dstb], add=True)
            return carry

        lax.fori_loop(0, _CHUNKS // 2, body, 0)
        plsc.subcore_barrier()
        pltpu.sync_copy(acc.at[pl.ds(s * _RPS, _RPS)],
                        out_hbm.at[c, pl.ds(s * _RPS, _RPS)])

    return _scat


_scatter_w = _make_scatter(_W, True)
_scatter_n = _make_scatter(_HQ, False)


@functools.partial(
    pl.kernel,
    out_type=jax.ShapeDtypeStruct((_B, _HQ), jnp.float32),
    mesh=_mesh,
    scratch_types=[
        pltpu.VMEM((_BPW,), jnp.int32),
        pltpu.VMEM((_BPW, _HQ), jnp.float32),
        pltpu.SemaphoreType.DMA,
    ],
    compiler_params=pltpu.CompilerParams(use_tc_tiling_on_sc=False),
)
def _gather_rows(h_hbm, idx_hbm, out_hbm, idx_v, rows_v, sem):
    c = lax.axis_index("c")
    s = lax.axis_index("s")
    wid = c * _NS + s
    pltpu.sync_copy(idx_hbm.at[pl.ds(wid * _BPW, _BPW)], idx_v)
    pltpu.async_copy(h_hbm.at[idx_v], rows_v, sem).wait()
    pltpu.sync_copy(rows_v, out_hbm.at[pl.ds(wid * _BPW, _BPW)])


def _scale_in_body(degp_ref, x_ref, w_ref, g_ref, dinv_ref):
    deg = jnp.sum(degp_ref[...], axis=0)[:, None] + 1.0
    dinv = lax.rsqrt(deg)
    i = pl.program_id(0)
    rows = i * _BLK + lax.broadcasted_iota(jnp.int32, (_BLK, 1), 0)
    h = jnp.dot(x_ref[...], w_ref[...], preferred_element_type=jnp.float32)
    g_ref[...] = jnp.where(rows < _N, h * dinv, 0.0)
    dinv_ref[...] = dinv


_scale_in = pl.pallas_call(
    _scale_in_body,
    grid=(_NP // _BLK,),
    in_specs=[
        pl.BlockSpec((_NW, _BLK), lambda i: (0, i)),
        pl.BlockSpec((_BLK, _F), lambda i: (i, 0)),
        pl.BlockSpec((_F, _W), lambda i: (0, 0)),
    ],
    out_specs=[
        pl.BlockSpec((_BLK, _W), lambda i: (i, 0)),
        pl.BlockSpec((_BLK, 1), lambda i: (i, 0)),
    ],
    out_shape=[
        jax.ShapeDtypeStruct((_NP, _W), jnp.float32),
        jax.ShapeDtypeStruct((_NP, 1), jnp.float32),
    ],
)


def _make_combine_matmul(din, dout):
    def body(p_ref, g_ref, dinv_ref, b_ref, w_ref, out_ref):
        i = pl.program_id(0)
        p = p_ref[...]
        dinv = dinv_ref[...]
        u = dinv * (p[0] + p[1] + g_ref[...]) + b_ref[...]
        rows = i * _BLK + lax.broadcasted_iota(jnp.int32, (_BLK, 1), 0)
        v = jnp.where(rows < _N, jnp.maximum(u, 0.0), 0.0)
        out_ref[...] = jnp.dot(
            v, w_ref[...], preferred_element_type=jnp.float32) * dinv

    return pl.pallas_call(
        body,
        grid=(_NP // _BLK,),
        in_specs=[
            pl.BlockSpec((2, _BLK, din), lambda i: (0, i, 0)),
            pl.BlockSpec((_BLK, din), lambda i: (i, 0)),
            pl.BlockSpec((_BLK, 1), lambda i: (i, 0)),
            pl.BlockSpec((1, din), lambda i: (0, 0)),
            pl.BlockSpec((din, dout), lambda i: (0, 0)),
        ],
        out_specs=pl.BlockSpec((_BLK, dout), lambda i: (i, 0)),
        out_shape=jax.ShapeDtypeStruct((_NP, dout), jnp.float32),
    )


_combine12 = _make_combine_matmul(_W, _HQ)
_combine23 = _make_combine_matmul(_HQ, _HQ)


def _combine_out_body(p_ref, g_ref, dinv_ref, b_ref, out_ref):
    p = p_ref[...]
    out_ref[...] = dinv_ref[...] * (p[0] + p[1] + g_ref[...]) + b_ref[...]


_combine_out = pl.pallas_call(
    _combine_out_body,
    grid=(_NP // _BLK,),
    in_specs=[
        pl.BlockSpec((2, _BLK, _HQ), lambda i: (0, i, 0)),
        pl.BlockSpec((_BLK, _HQ), lambda i: (i, 0)),
        pl.BlockSpec((_BLK, 1), lambda i: (i, 0)),
        pl.BlockSpec((1, _HQ), lambda i: (0, 0)),
    ],
    out_specs=pl.BlockSpec((_BLK, _HQ), lambda i: (i, 0)),
    out_shape=jax.ShapeDtypeStruct((_NP, _HQ), jnp.float32),
)


def _head_body(hg_ref, wl_ref, bl_ref, lsm_ref, sm_ref):
    z = jnp.dot(hg_ref[...], wl_ref[...],
                preferred_element_type=jnp.float32) + bl_ref[...]
    m = jnp.max(z, axis=1, keepdims=True)
    zc = z - m
    e = jnp.exp(zc)
    ssum = jnp.sum(e, axis=1, keepdims=True)
    sm_ref[...] = e / ssum
    lsm_ref[...] = zc - jnp.log(ssum)


_head = pl.pallas_call(
    _head_body,
    out_shape=[
        jax.ShapeDtypeStruct((_B, 2), jnp.float32),
        jax.ShapeDtypeStruct((_B, 2), jnp.float32),
    ],
)


def kernel(x, edge_index, relevant_batch_indices, labels,
           W1, b1, W2, b2, W3, b3, Wlin, blin):
    src = jnp.asarray(edge_index[0], jnp.int32)
    dst = jnp.asarray(edge_index[1], jnp.int32)
    padv = jnp.full((_NE - _E,), _N, jnp.int32)
    src_p = jnp.concatenate([src, padv])
    dst_p = jnp.concatenate([dst, padv])
    x_p = jnp.pad(x, ((0, _NP - _N), (0, 0)))

    src3 = src_p.reshape(_NW, _CHUNKS, _CH)
    dst3 = dst_p.reshape(_NW, _CHUNKS, _CH)

    degp = _deg_kernel(dst_p)
    g1, dinv = _scale_in(degp, x_p, W1)
    p1 = _scatter_w(g1, src3, dst3)
    g2 = _combine12(p1, g1, dinv, b1.reshape(1, -1), W2)
    p2 = _scatter_n(g2, src3, dst3)
    g3 = _combine23(p2, g2, dinv, b2.reshape(1, -1), W3)
    p3 = _scatter_n(g3, src3, dst3)
    h3 = _combine_out(p3, g3, dinv, b3.reshape(1, -1))
    hg = _gather_rows(h3, jnp.asarray(relevant_batch_indices, jnp.int32))
    lsm, sm = _head(hg, Wlin, blin.reshape(1, -1))
    return (lsm, sm)
